# baseline (speedup 1.0000x reference)
import functools

import jax
import jax.numpy as jnp
from jax import lax
from jax.experimental import pallas as pl
from jax.experimental.pallas import tpu as pltpu
from jax.experimental.pallas import tpu_sc as plsc

B = 4 * 8192
D = 512
NW = 32
BPW = B // NW
CHUNK = 32
NCHUNK = BPW // CHUNK
NB = 6

_mesh = plsc.VectorSubcoreMesh(core_axis_name="c", subcore_axis_name="s")


@functools.partial(
    pl.kernel,
    mesh=_mesh,
    out_type=jax.ShapeDtypeStruct((B, D), jnp.float32),
    scratch_types=(
        [pltpu.VMEM((NCHUNK, CHUNK), jnp.int32)]
        + [pltpu.VMEM((CHUNK, D), jnp.float32)] * NB
        + [pltpu.SemaphoreType.DMA] * (2 * NB)
    ),
)
def _gather_kernel(idx_hbm, table_hbm, out_hbm, idx_v, *rest):
    bufs = rest[:NB]
    gsems = rest[NB:2 * NB]
    osems = rest[2 * NB:]
    wid = lax.axis_index("s") * 2 + lax.axis_index("c")
    base = wid * BPW

    pltpu.sync_copy(idx_hbm.at[wid], idx_v)

    def start_gather(j, b):
        return pltpu.async_copy(table_hbm.at[idx_v.at[j]], bufs[b], gsems[b])

    pending_g = [None] * NB
    pending_o = [None] * NB
    for j in range(NB - 1):
        pending_g[j] = start_gather(j, j)
    for j in range(NCHUNK):
        b = j % NB
        nb = (j + NB - 1) % NB
        if j + NB - 1 < NCHUNK:
            if pending_o[nb] is not None:
                pending_o[nb].wait()
                pending_o[nb] = None
            pending_g[nb] = start_gather(j + NB - 1, nb)
        pending_g[b].wait()
        pending_o[b] = pltpu.async_copy(
            bufs[b], out_hbm.at[pl.ds(base + j * CHUNK, CHUNK)], osems[b])
    for b in range(NB):
        if pending_o[b] is not None:
            pending_o[b].wait()


def kernel(src_pos, position_enc_weight):
    idx = src_pos.reshape(NW, NCHUNK, CHUNK)
    out = _gather_kernel(idx, position_enc_weight)
    return out.reshape(src_pos.shape + (D,))


# speedup vs baseline: 1.0018x; 1.0018x over previous
import functools

import jax
import jax.numpy as jnp
from jax import lax
from jax.experimental import pallas as pl
from jax.experimental.pallas import tpu as pltpu
from jax.experimental.pallas import tpu_sc as plsc

BATCH = 4
SEQ = 8192
D = 512
NW = 32
WPB = NW // BATCH
BPW = SEQ // WPB
CHUNK = 64
NCHUNK = BPW // CHUNK
NB = 3

_mesh = plsc.VectorSubcoreMesh(core_axis_name="c", subcore_axis_name="s")


@functools.partial(
    pl.kernel,
    mesh=_mesh,
    out_type=jax.ShapeDtypeStruct((BATCH, SEQ, D), jnp.float32),
    scratch_types=(
        [pltpu.VMEM((BPW,), jnp.int32)]
        + [pltpu.VMEM((CHUNK, D), jnp.float32)] * NB
        + [pltpu.SemaphoreType.DMA] * (2 * NB)
    ),
)
def _gather_kernel(idx_hbm, table_hbm, out_hbm, idx_v, *rest):
    bufs = rest[:NB]
    gsems = rest[NB:2 * NB]
    osems = rest[2 * NB:]
    wid = lax.axis_index("s") * 2 + lax.axis_index("c")
    row = wid // WPB
    col = (wid % WPB) * BPW

    pltpu.sync_copy(idx_hbm.at[row, pl.ds(col, BPW)], idx_v)

    def start_gather(j, b):
        return pltpu.async_copy(
            table_hbm.at[idx_v.at[pl.ds(j * CHUNK, CHUNK)]], bufs[b],
            gsems[b])

    pending_g = [None] * NB
    pending_o = [None] * NB
    for j in range(NB - 1):
        pending_g[j] = start_gather(j, j)
    for j in range(NCHUNK):
        b = j % NB
        nb = (j + NB - 1) % NB
        if j + NB - 1 < NCHUNK:
            if pending_o[nb] is not None:
                pending_o[nb].wait()
                pending_o[nb] = None
            pending_g[nb] = start_gather(j + NB - 1, nb)
        pending_g[b].wait()
        pending_o[b] = pltpu.async_copy(
            bufs[b], out_hbm.at[row, pl.ds(col + j * CHUNK, CHUNK)],
            osems[b])
    for b in range(NB):
        if pending_o[b] is not None:
            pending_o[b].wait()


def kernel(src_pos, position_enc_weight):
    return _gather_kernel(src_pos, position_enc_weight)
